# baseline (speedup 1.0000x reference)
import functools

import jax
import jax.numpy as jnp
from jax import lax
from jax.experimental import pallas as pl
from jax.experimental.pallas import tpu as pltpu
from jax.experimental.pallas import tpu_sc as plsc

_B = 1024
_IN = 512
_H = 128
_K = 2000
_KP = 2048
_TOPK = 5

_B_SC = 256
_B_TC = _B - _B_SC

_KG = 8
_BG = 4
_KS = _KP // _KG
_BC = 32
_UKT = 4
_L = 16


def _fc1_body(x_ref, w1t_ref, b1_ref, h_ref):
    h_ref[...] = jnp.tanh(
        jnp.dot(x_ref[...], w1t_ref[...], preferred_element_type=jnp.float32)
        + b1_ref[...]
    )


_fc1 = pl.pallas_call(
    _fc1_body,
    out_shape=jax.ShapeDtypeStruct((_B, _H), jnp.float32),
)


def _fc2_body(y_ref, w2t_ref, b2_ref, o_ref):
    o_ref[...] = (
        jnp.dot(y_ref[...], w2t_ref[...], preferred_element_type=jnp.float32)
        + b2_ref[...]
    )


def _fc2(y, w2t, b2r):
    return pl.pallas_call(
        _fc2_body,
        out_shape=jax.ShapeDtypeStruct((y.shape[0], 3), jnp.float32),
    )(y, w2t, b2r)


def _kenyon_sc_body(h_hbm, wkt_hbm, y_hbm, h_v, wk_v, out_v):
    bs = _B_SC // _BG
    wid = lax.axis_index("s") * 2 + lax.axis_index("c")
    kg = wid % _KG
    bg = wid // _KG
    k0 = kg * _KS
    b0 = bg * bs

    pltpu.sync_copy(wkt_hbm.at[:, pl.ds(k0, _KS)], wk_v)
    pltpu.sync_copy(h_hbm.at[pl.ds(b0, bs), :], h_v)

    neg = jnp.full((_L,), -jnp.inf, dtype=jnp.float32)

    def chunk_loop(c, carry):
        def b_loop(bi, carry):
            b = c * _BC + bi

            def ktg_loop(ktg, carry):
                def jc_loop(jc, ms):
                    hv = h_v[b, pl.ds(jc * _L, _L)]
                    ms = list(ms)
                    for tj in range(_L):
                        hs = hv[tj]
                        for u in range(_UKT):
                            w = wk_v[
                                jc * _L + tj,
                                pl.ds((ktg * _UKT + u) * _L, _L),
                            ]
                            v = hs * w
                            for t in range(_TOPK):
                                idx = u * _TOPK + t
                                nt = jnp.maximum(ms[idx], v)
                                if t < _TOPK - 1:
                                    v = jnp.minimum(ms[idx], v)
                                ms[idx] = nt
                    return tuple(ms)

                ms = lax.fori_loop(
                    0, _H // _L, jc_loop, (neg,) * (_TOPK * _UKT)
                )
                for u in range(_UKT):
                    s = ms[u * _TOPK]
                    for t in range(1, _TOPK):
                        s = s + ms[u * _TOPK + t]
                    out_v[bi, pl.ds((ktg * _UKT + u) * _L, _L)] = s
                return carry

            return lax.fori_loop(0, _KS // _L // _UKT, ktg_loop, carry)

        lax.fori_loop(0, _BC, b_loop, carry)
        pltpu.sync_copy(
            out_v, y_hbm.at[pl.ds(b0 + c * _BC, _BC), pl.ds(k0, _KS)]
        )
        return carry

    lax.fori_loop(0, _B_SC // _BG // _BC, chunk_loop, 0)


@functools.cache
def _build_kenyon_sc():
    sc_mesh = plsc.VectorSubcoreMesh(
        core_axis_name="c", subcore_axis_name="s", num_cores=2, num_subcores=16
    )
    return pl.kernel(
        _kenyon_sc_body,
        out_type=jax.ShapeDtypeStruct((_B_SC, _KP), jnp.float32),
        mesh=sc_mesh,
        scratch_types=[
            pltpu.VMEM((_B_SC // _BG, _H), jnp.float32),
            pltpu.VMEM((_H, _KS), jnp.float32),
            pltpu.VMEM((_BC, _KS), jnp.float32),
        ],
    )


def _kenyon_tc_body(hrep_ref, wkt_ref, o_ref):
    neg = jnp.full((8, 128), -jnp.inf, dtype=jnp.float32)
    for kt in range(_KP // 128):

        def jc_body(jc, ms):
            hchunk = hrep_ref[pl.ds(pl.multiple_of(jc * 64, 64), 64), :]
            wchunk = wkt_ref[
                pl.ds(pl.multiple_of(jc * 8, 8), 8), pl.ds(kt * 128, 128)
            ]
            m0, m1, m2, m3, m4 = ms
            for t in range(8):
                hb = hchunk[t * 8:(t + 1) * 8, :]
                v = hb * wchunk[t:t + 1, :]
                n0 = jnp.maximum(m0, v)
                v = jnp.minimum(m0, v)
                n1 = jnp.maximum(m1, v)
                v = jnp.minimum(m1, v)
                n2 = jnp.maximum(m2, v)
                v = jnp.minimum(m2, v)
                n3 = jnp.maximum(m3, v)
                v = jnp.minimum(m3, v)
                n4 = jnp.maximum(m4, v)
                m0, m1, m2, m3, m4 = n0, n1, n2, n3, n4
            return (m0, m1, m2, m3, m4)

        ms = lax.fori_loop(0, _H // 8, jc_body, (neg,) * _TOPK)
        o_ref[:, pl.ds(kt * 128, 128)] = (
            ((ms[0] + ms[1]) + (ms[2] + ms[3])) + ms[4]
        )


_kenyon_tc = pl.pallas_call(
    _kenyon_tc_body,
    grid=(_B_TC // 8,),
    in_specs=[
        pl.BlockSpec((_H * 8, _H), lambda i: (i, 0)),
        pl.BlockSpec((_H, _KP), lambda i: (0, 0)),
    ],
    out_specs=pl.BlockSpec((8, _KP), lambda i: (i, 0)),
    out_shape=jax.ShapeDtypeStruct((_B_TC, _KP), jnp.float32),
)


def _lane_broadcast_h(h_tc):
    g = _B_TC // 8
    hre = h_tc.reshape(g, 8, _H).transpose(0, 2, 1)
    hrep = jnp.broadcast_to(hre[:, :, :, None], (g, _H, 8, _H))
    return hrep.reshape(g * _H * 8, _H)


def kernel(x, W1, b1, Wk, W2, b2):
    h = _fc1(x, W1.T, b1.reshape(1, _H))
    wkt = jnp.pad(Wk, ((0, _KP - _K), (0, 0))).T
    y_sc = _build_kenyon_sc()(h[:_B_SC], wkt)
    y_tc = _kenyon_tc(_lane_broadcast_h(h[_B_SC:]), wkt)
    w2t = jnp.pad(W2, ((0, 0), (0, _KP - _K))).T
    b2r = b2.reshape(1, 3)
    return jnp.concatenate([_fc2(y_sc, w2t, b2r), _fc2(y_tc, w2t, b2r)], axis=0)


# speedup vs baseline: 1.1348x; 1.1348x over previous
import functools

import jax
import jax.numpy as jnp
from jax import lax
from jax.experimental import pallas as pl
from jax.experimental.pallas import tpu as pltpu
from jax.experimental.pallas import tpu_sc as plsc

_B = 1024
_IN = 512
_H = 128
_K = 2000
_KP = 2048
_TOPK = 5

_B_SC = 384
_B_TC = _B - _B_SC

_KG = 8
_BG = 4
_KS = _KP // _KG
_BC = 32
_UKT = 4
_L = 16


def _fc1_body(x_ref, w1t_ref, b1_ref, h_ref):
    h_ref[...] = jnp.tanh(
        jnp.dot(x_ref[...], w1t_ref[...], preferred_element_type=jnp.float32)
        + b1_ref[...]
    )


_fc1 = pl.pallas_call(
    _fc1_body,
    out_shape=jax.ShapeDtypeStruct((_B, _H), jnp.float32),
)


def _fc2_body(y_ref, w2t_ref, b2_ref, o_ref):
    o_ref[...] = (
        jnp.dot(y_ref[...], w2t_ref[...], preferred_element_type=jnp.float32)
        + b2_ref[...]
    )


def _fc2(y, w2t, b2r):
    return pl.pallas_call(
        _fc2_body,
        out_shape=jax.ShapeDtypeStruct((y.shape[0], 3), jnp.float32),
    )(y, w2t, b2r)


def _kenyon_sc_body(h_hbm, wkt_hbm, y_hbm, h_v, wk_v, out_v):
    bs = _B_SC // _BG
    wid = lax.axis_index("s") * 2 + lax.axis_index("c")
    kg = wid % _KG
    bg = wid // _KG
    k0 = kg * _KS
    b0 = bg * bs

    pltpu.sync_copy(wkt_hbm.at[:, pl.ds(k0, _KS)], wk_v)
    pltpu.sync_copy(h_hbm.at[pl.ds(b0, bs), :], h_v)

    neg = jnp.full((_L,), -jnp.inf, dtype=jnp.float32)

    def chunk_loop(c, carry):
        def b_loop(bi, carry):
            b = c * _BC + bi

            def ktg_loop(ktg, carry):
                def jc_loop(jc, ms):
                    hv = h_v[b, pl.ds(jc * _L, _L)]
                    ms = list(ms)
                    for tj in range(_L):
                        hs = hv[tj]
                        for u in range(_UKT):
                            w = wk_v[
                                jc * _L + tj,
                                pl.ds((ktg * _UKT + u) * _L, _L),
                            ]
                            v = hs * w
                            for t in range(_TOPK):
                                idx = u * _TOPK + t
                                nt = jnp.maximum(ms[idx], v)
                                if t < _TOPK - 1:
                                    v = jnp.minimum(ms[idx], v)
                                ms[idx] = nt
                    return tuple(ms)

                ms = lax.fori_loop(
                    0, _H // _L, jc_loop, (neg,) * (_TOPK * _UKT)
                )
                for u in range(_UKT):
                    s = ms[u * _TOPK]
                    for t in range(1, _TOPK):
                        s = s + ms[u * _TOPK + t]
                    out_v[bi, pl.ds((ktg * _UKT + u) * _L, _L)] = s
                return carry

            return lax.fori_loop(0, _KS // _L // _UKT, ktg_loop, carry)

        lax.fori_loop(0, _BC, b_loop, carry)
        pltpu.sync_copy(
            out_v, y_hbm.at[pl.ds(b0 + c * _BC, _BC), pl.ds(k0, _KS)]
        )
        return carry

    lax.fori_loop(0, _B_SC // _BG // _BC, chunk_loop, 0)


@functools.cache
def _build_kenyon_sc():
    sc_mesh = plsc.VectorSubcoreMesh(
        core_axis_name="c", subcore_axis_name="s", num_cores=2, num_subcores=16
    )
    return pl.kernel(
        _kenyon_sc_body,
        out_type=jax.ShapeDtypeStruct((_B_SC, _KP), jnp.float32),
        mesh=sc_mesh,
        scratch_types=[
            pltpu.VMEM((_B_SC // _BG, _H), jnp.float32),
            pltpu.VMEM((_H, _KS), jnp.float32),
            pltpu.VMEM((_BC, _KS), jnp.float32),
        ],
    )


def _kenyon_tc_body(hrep_ref, wkt_ref, o_ref):
    neg = jnp.full((8, 128), -jnp.inf, dtype=jnp.float32)
    for kt in range(_KP // 128):

        def jc_body(jc, ms):
            hchunk = hrep_ref[pl.ds(pl.multiple_of(jc * 64, 64), 64), :]
            wchunk = wkt_ref[
                pl.ds(pl.multiple_of(jc * 8, 8), 8), pl.ds(kt * 128, 128)
            ]
            m0, m1, m2, m3, m4 = ms
            for t in range(8):
                hb = hchunk[t * 8:(t + 1) * 8, :]
                v = hb * wchunk[t:t + 1, :]
                n0 = jnp.maximum(m0, v)
                v = jnp.minimum(m0, v)
                n1 = jnp.maximum(m1, v)
                v = jnp.minimum(m1, v)
                n2 = jnp.maximum(m2, v)
                v = jnp.minimum(m2, v)
                n3 = jnp.maximum(m3, v)
                v = jnp.minimum(m3, v)
                n4 = jnp.maximum(m4, v)
                m0, m1, m2, m3, m4 = n0, n1, n2, n3, n4
            return (m0, m1, m2, m3, m4)

        ms = lax.fori_loop(0, _H // 8, jc_body, (neg,) * _TOPK)
        o_ref[:, pl.ds(kt * 128, 128)] = (
            ((ms[0] + ms[1]) + (ms[2] + ms[3])) + ms[4]
        )


_kenyon_tc = pl.pallas_call(
    _kenyon_tc_body,
    grid=(_B_TC // 8,),
    in_specs=[
        pl.BlockSpec((_H * 8, _H), lambda i: (i, 0)),
        pl.BlockSpec((_H, _KP), lambda i: (0, 0)),
    ],
    out_specs=pl.BlockSpec((8, _KP), lambda i: (i, 0)),
    out_shape=jax.ShapeDtypeStruct((_B_TC, _KP), jnp.float32),
)


def _lane_broadcast_h(h_tc):
    g = _B_TC // 8
    hre = h_tc.reshape(g, 8, _H).transpose(0, 2, 1)
    hrep = jnp.broadcast_to(hre[:, :, :, None], (g, _H, 8, _H))
    return hrep.reshape(g * _H * 8, _H)


def kernel(x, W1, b1, Wk, W2, b2):
    h = _fc1(x, W1.T, b1.reshape(1, _H))
    wkt = jnp.pad(Wk, ((0, _KP - _K), (0, 0))).T
    y_sc = _build_kenyon_sc()(h[:_B_SC], wkt)
    y_tc = _kenyon_tc(_lane_broadcast_h(h[_B_SC:]), wkt)
    w2t = jnp.pad(W2, ((0, 0), (0, _KP - _K))).T
    b2r = b2.reshape(1, 3)
    return jnp.concatenate([_fc2(y_sc, w2t, b2r), _fc2(y_tc, w2t, b2r)], axis=0)


# speedup vs baseline: 1.1576x; 1.0201x over previous
import functools

import jax
import jax.numpy as jnp
from jax import lax
from jax.experimental import pallas as pl
from jax.experimental.pallas import tpu as pltpu
from jax.experimental.pallas import tpu_sc as plsc

_B = 1024
_IN = 512
_H = 128
_K = 2000
_KP = 2048
_TOPK = 5

_B_SC = 384
_B_TC = _B - _B_SC

_KG = 8
_BG = 4
_KS = _KP // _KG
_BC = 32
_UKT = 4
_L = 16


def _fc1_body(x_ref, w1t_ref, b1_ref, h_ref):
    h_ref[...] = jnp.tanh(
        jnp.dot(x_ref[...], w1t_ref[...], preferred_element_type=jnp.float32)
        + b1_ref[...]
    )


_fc1 = pl.pallas_call(
    _fc1_body,
    out_shape=jax.ShapeDtypeStruct((_B, _H), jnp.float32),
)


def _fc2_body(y_ref, w2t_ref, b2_ref, o_ref):
    o_ref[...] = (
        jnp.dot(y_ref[...], w2t_ref[...], preferred_element_type=jnp.float32)
        + b2_ref[...]
    )


def _fc2(y, w2t, b2r):
    return pl.pallas_call(
        _fc2_body,
        out_shape=jax.ShapeDtypeStruct((y.shape[0], 3), jnp.float32),
    )(y, w2t, b2r)


def _kenyon_sc_body(h_hbm, wkt_hbm, y_hbm, h_v, wk_v, out_v):
    bs = _B_SC // _BG
    wid = lax.axis_index("s") * 2 + lax.axis_index("c")
    kg = wid % _KG
    bg = wid // _KG
    k0 = kg * _KS
    b0 = bg * bs

    pltpu.sync_copy(wkt_hbm.at[:, pl.ds(k0, _KS)], wk_v)
    pltpu.sync_copy(h_hbm.at[pl.ds(b0, bs), :], h_v)

    neg = jnp.full((_L,), -jnp.inf, dtype=jnp.float32)

    def chunk_loop(c, carry):
        def b_loop(bi, carry):
            b = c * _BC + bi

            def ktg_loop(ktg, carry):
                def jc_loop(jc, ms):
                    hv = h_v[b, pl.ds(jc * _L, _L)]
                    ms = list(ms)
                    for tj in range(_L):
                        hs = hv[tj]
                        for u in range(_UKT):
                            w = wk_v[
                                jc * _L + tj,
                                pl.ds((ktg * _UKT + u) * _L, _L),
                            ]
                            v = hs * w
                            for t in range(_TOPK):
                                idx = u * _TOPK + t
                                nt = jnp.maximum(ms[idx], v)
                                if t < _TOPK - 1:
                                    v = jnp.minimum(ms[idx], v)
                                ms[idx] = nt
                    return tuple(ms)

                ms = lax.fori_loop(
                    0, _H // _L, jc_loop, (neg,) * (_TOPK * _UKT)
                )
                for u in range(_UKT):
                    s = ms[u * _TOPK]
                    for t in range(1, _TOPK):
                        s = s + ms[u * _TOPK + t]
                    out_v[bi, pl.ds((ktg * _UKT + u) * _L, _L)] = s
                return carry

            return lax.fori_loop(0, _KS // _L // _UKT, ktg_loop, carry)

        lax.fori_loop(0, _BC, b_loop, carry)
        pltpu.sync_copy(
            out_v, y_hbm.at[pl.ds(b0 + c * _BC, _BC), pl.ds(k0, _KS)]
        )
        return carry

    lax.fori_loop(0, _B_SC // _BG // _BC, chunk_loop, 0)


@functools.cache
def _build_kenyon_sc():
    sc_mesh = plsc.VectorSubcoreMesh(
        core_axis_name="c", subcore_axis_name="s", num_cores=2, num_subcores=16
    )
    return pl.kernel(
        _kenyon_sc_body,
        out_type=jax.ShapeDtypeStruct((_B_SC, _KP), jnp.float32),
        mesh=sc_mesh,
        scratch_types=[
            pltpu.VMEM((_B_SC // _BG, _H), jnp.float32),
            pltpu.VMEM((_H, _KS), jnp.float32),
            pltpu.VMEM((_BC, _KS), jnp.float32),
        ],
    )


_TB = 16


def _kenyon_tc_body(hrep_ref, wkt_ref, o_ref):
    neg = jnp.full((_TB, 128), -jnp.inf, dtype=jnp.bfloat16)
    for kt in range(_KP // 128):

        def jc_body(jc, ms):
            hchunk = hrep_ref[pl.ds(pl.multiple_of(jc * _TB * 8, _TB * 8), _TB * 8), :]
            wchunk = wkt_ref[
                pl.ds(pl.multiple_of(jc * 8, 8), 8), pl.ds(kt * 128, 128)
            ]
            m0, m1, m2, m3, m4 = ms
            for t in range(8):
                hb = hchunk[t * _TB:(t + 1) * _TB, :]
                v = hb * wchunk[t:t + 1, :]
                n0 = jnp.maximum(m0, v)
                v = jnp.minimum(m0, v)
                n1 = jnp.maximum(m1, v)
                v = jnp.minimum(m1, v)
                n2 = jnp.maximum(m2, v)
                v = jnp.minimum(m2, v)
                n3 = jnp.maximum(m3, v)
                v = jnp.minimum(m3, v)
                n4 = jnp.maximum(m4, v)
                m0, m1, m2, m3, m4 = n0, n1, n2, n3, n4
            return (m0, m1, m2, m3, m4)

        ms = lax.fori_loop(0, _H // 8, jc_body, (neg,) * _TOPK)
        o_ref[:, pl.ds(kt * 128, 128)] = (
            ((ms[0] + ms[1]) + (ms[2] + ms[3])) + ms[4]
        ).astype(jnp.float32)


_kenyon_tc = pl.pallas_call(
    _kenyon_tc_body,
    grid=(_B_TC // _TB,),
    in_specs=[
        pl.BlockSpec((_H * _TB, _H), lambda i: (i, 0)),
        pl.BlockSpec((_H, _KP), lambda i: (0, 0)),
    ],
    out_specs=pl.BlockSpec((_TB, _KP), lambda i: (i, 0)),
    out_shape=jax.ShapeDtypeStruct((_B_TC, _KP), jnp.float32),
)


def _lane_broadcast_h(h_tc):
    g = _B_TC // _TB
    hre = h_tc.astype(jnp.bfloat16).reshape(g, _TB, _H).transpose(0, 2, 1)
    hrep = jnp.broadcast_to(hre[:, :, :, None], (g, _H, _TB, _H))
    return hrep.reshape(g * _H * _TB, _H)


def kernel(x, W1, b1, Wk, W2, b2):
    h = _fc1(x, W1.T, b1.reshape(1, _H))
    wkt = jnp.pad(Wk, ((0, _KP - _K), (0, 0))).T
    y_sc = _build_kenyon_sc()(h[:_B_SC], wkt)
    y_tc = _kenyon_tc(
        _lane_broadcast_h(h[_B_SC:]), wkt.astype(jnp.bfloat16)
    )
    w2t = jnp.pad(W2, ((0, 0), (0, _KP - _K))).T
    b2r = b2.reshape(1, 3)
    return jnp.concatenate([_fc2(y_sc, w2t, b2r), _fc2(y_tc, w2t, b2r)], axis=0)
